# baseline (speedup 1.0000x reference)
import functools

import jax
import jax.numpy as jnp
from jax import lax
from jax.experimental import pallas as pl
from jax.experimental.pallas import tpu as pltpu
from jax.experimental.pallas import tpu_sc as plsc

EMBED_DIM = 64
PAD_DIM = 128
NC = 2
NS = 16
NW = NC * NS
LANES = 16

_CW = 256
_NB = 128
_NBUF = 2


def _transpose_16(src_ref, dst_ref, n, scatter):
    steps = n // LANES
    for d in range(EMBED_DIM):
        def body(kk, carry):
            rows = kk * LANES + lax.iota(jnp.int32, LANES)
            cols = jnp.full((LANES,), d, jnp.int32)
            if scatter:
                v = src_ref[d, pl.ds(kk * LANES, LANES)]
                plsc.store_scatter(dst_ref, [rows, cols], v)
            else:
                v = plsc.load_gather(src_ref, [rows, cols])
                dst_ref[d, pl.ds(kk * LANES, LANES)] = v
            return carry

        lax.fori_loop(0, steps, body, 0)


def _make_format_table(V):
    n_full = V // _CW
    tail = V - n_full * _CW
    n_iters = (n_full + NW - 1) // NW

    mesh = plsc.VectorSubcoreMesh(core_axis_name="c", subcore_axis_name="s")

    @functools.partial(
        pl.kernel,
        mesh=mesh,
        out_type=jax.ShapeDtypeStruct((V, PAD_DIM), jnp.float32),
        scratch_types=[
            pltpu.VMEM((EMBED_DIM, _CW), jnp.float32),
            pltpu.VMEM((_CW, PAD_DIM), jnp.float32),
            pltpu.VMEM((EMBED_DIM, tail), jnp.float32),
        ],
        compiler_params=pltpu.CompilerParams(use_tc_tiling_on_sc=True, needs_layout_passes=False),
    )
    def tpose_kernel(tt_hbm, out_hbm, tsrc, tdst, ttail):
        wid = lax.axis_index("s") * NC + lax.axis_index("c")

        def chunk(k, carry):
            c = k * NW + wid

            @pl.when(c < n_full)
            def _():
                c0 = pl.multiple_of(c * _CW, _CW)
                pltpu.sync_copy(tt_hbm.at[:, pl.ds(c0, _CW)], tsrc)
                _transpose_16(tsrc, tdst, _CW, scatter=True)
                pltpu.sync_copy(tdst, out_hbm.at[pl.ds(c0, _CW)])
            return carry

        lax.fori_loop(0, n_iters, chunk, 0)

        @pl.when(wid == 0)
        def _():
            c0 = n_full * _CW
            pltpu.sync_copy(tt_hbm.at[:, pl.ds(c0, tail)], ttail)
            _transpose_16(ttail, tdst, tail, scatter=True)
            pltpu.sync_copy(
                tdst.at[pl.ds(0, tail)], out_hbm.at[pl.ds(c0, tail)]
            )

    return tpose_kernel


def _make_gather(N, S):
    mesh = plsc.VectorSubcoreMesh(core_axis_name="c", subcore_axis_name="s")

    @functools.partial(
        pl.kernel,
        mesh=mesh,
        out_type=jax.ShapeDtypeStruct((S, EMBED_DIM, N), jnp.float32),
        scratch_types=[
            pltpu.VMEM((S, _NB), jnp.int32),
            pltpu.VMEM((_NBUF, _NB, PAD_DIM), jnp.float32),
            pltpu.VMEM((_NBUF, EMBED_DIM, _NB), jnp.float32),
            pltpu.SemaphoreType.DMA((_NBUF,)),
            pltpu.SemaphoreType.DMA((_NBUF,)),
        ],
        compiler_params=pltpu.CompilerParams(use_tc_tiling_on_sc=True, needs_layout_passes=False),
    )
    def gather_kernel(tab_hbm, idx_hbm, out_hbm, idx_v, rows_v, obuf, gsem, ssem):
        wid = lax.axis_index("s") * NC + lax.axis_index("c")
        n0 = pl.multiple_of(wid * _NB, _NB)
        pltpu.sync_copy(idx_hbm.at[:, pl.ds(n0, _NB)], idx_v)

        def fire_gather(r, b):
            pltpu.async_copy(
                tab_hbm.at[idx_v.at[r]], rows_v.at[b], gsem.at[b]
            )

        def wait_gather(b):
            pltpu.make_async_copy(
                tab_hbm.at[pl.ds(0, _NB)], rows_v.at[b], gsem.at[b]
            ).wait()

        def fire_store(r, b):
            pltpu.async_copy(
                obuf.at[b], out_hbm.at[r].at[:, pl.ds(n0, _NB)], ssem.at[b]
            )

        def wait_store(b):
            pltpu.make_async_copy(
                obuf.at[b], out_hbm.at[0].at[:, pl.ds(0, _NB)], ssem.at[b]
            ).wait()

        def flush(r, b):
            _transpose_16(rows_v.at[b], obuf.at[b], _NB, scatter=False)
            fire_store(r, b)

        def step(k, carry):
            fire_gather(2 * k, 0)

            @pl.when(k > 0)
            def _():
                wait_gather(1)

            @pl.when(k > 1)
            def _():
                wait_store(1)

            @pl.when(k > 0)
            def _():
                flush(2 * k - 1, 1)

            fire_gather(2 * k + 1, 1)
            wait_gather(0)

            @pl.when(k > 0)
            def _():
                wait_store(0)

            flush(2 * k, 0)
            return carry

        lax.fori_loop(0, S // _NBUF, step, 0)
        wait_gather(1)
        wait_store(1)
        flush(S - 1, 1)
        wait_store(0)
        wait_store(1)

    return gather_kernel


@functools.partial(jax.jit, static_argnums=(2, 3, 4))
def _embed(table, idx, V, N, S):
    table_pad = _make_format_table(V)(table.T)
    out_t = _make_gather(N, S)(table_pad, idx.T)
    return out_t.transpose(2, 0, 1)


def kernel(pre_embedding, preembed_mask, embed_table):
    N, S = pre_embedding.shape
    V = embed_table.shape[0]
    out = _embed(embed_table, pre_embedding, V, N, S)
    return out, preembed_mask


# speedup vs baseline: 1.0265x; 1.0265x over previous
import functools

import jax
import jax.numpy as jnp
from jax import lax
from jax.experimental import pallas as pl
from jax.experimental.pallas import tpu as pltpu
from jax.experimental.pallas import tpu_sc as plsc

EMBED_DIM = 64
PAD_DIM = 128
NC = 2
NS = 16
NW = NC * NS
LANES = 16

_CW = 256
_NB = 128
_NBUF = 2


def _transpose_16(src_ref, dst_ref, n, scatter):
    steps = n // LANES

    def body(kk, carry):
        rows = kk * LANES + lax.iota(jnp.int32, LANES)
        for d in range(EMBED_DIM):
            cols = jnp.full((LANES,), d, jnp.int32)
            if scatter:
                v = src_ref[d, pl.ds(kk * LANES, LANES)]
                plsc.store_scatter(dst_ref, [rows, cols], v)
            else:
                v = plsc.load_gather(src_ref, [rows, cols])
                dst_ref[d, pl.ds(kk * LANES, LANES)] = v
        return carry

    lax.fori_loop(0, steps, body, 0)


def _make_format_table(V):
    n_full = V // _CW
    tail = V - n_full * _CW
    n_iters = (n_full + NW - 1) // NW

    mesh = plsc.VectorSubcoreMesh(core_axis_name="c", subcore_axis_name="s")

    @functools.partial(
        pl.kernel,
        mesh=mesh,
        out_type=jax.ShapeDtypeStruct((V, PAD_DIM), jnp.float32),
        scratch_types=[
            pltpu.VMEM((EMBED_DIM, _CW), jnp.float32),
            pltpu.VMEM((_CW, PAD_DIM), jnp.float32),
            pltpu.VMEM((EMBED_DIM, tail), jnp.float32),
        ],
        compiler_params=pltpu.CompilerParams(use_tc_tiling_on_sc=True, needs_layout_passes=False),
    )
    def tpose_kernel(tt_hbm, out_hbm, tsrc, tdst, ttail):
        wid = lax.axis_index("s") * NC + lax.axis_index("c")

        def chunk(k, carry):
            c = k * NW + wid

            @pl.when(c < n_full)
            def _():
                c0 = pl.multiple_of(c * _CW, _CW)
                pltpu.sync_copy(tt_hbm.at[:, pl.ds(c0, _CW)], tsrc)
                _transpose_16(tsrc, tdst, _CW, scatter=True)
                pltpu.sync_copy(tdst, out_hbm.at[pl.ds(c0, _CW)])
            return carry

        lax.fori_loop(0, n_iters, chunk, 0)

        @pl.when(wid == 0)
        def _():
            c0 = n_full * _CW
            pltpu.sync_copy(tt_hbm.at[:, pl.ds(c0, tail)], ttail)
            _transpose_16(ttail, tdst, tail, scatter=True)
            pltpu.sync_copy(
                tdst.at[pl.ds(0, tail)], out_hbm.at[pl.ds(c0, tail)]
            )

    return tpose_kernel


def _make_gather(N, S):
    mesh = plsc.VectorSubcoreMesh(core_axis_name="c", subcore_axis_name="s")

    @functools.partial(
        pl.kernel,
        mesh=mesh,
        out_type=jax.ShapeDtypeStruct((S, EMBED_DIM, N), jnp.float32),
        scratch_types=[
            pltpu.VMEM((S, _NB), jnp.int32),
            pltpu.VMEM((_NBUF, _NB, PAD_DIM), jnp.float32),
            pltpu.VMEM((_NBUF, EMBED_DIM, _NB), jnp.float32),
            pltpu.SemaphoreType.DMA((_NBUF,)),
            pltpu.SemaphoreType.DMA((_NBUF,)),
        ],
        compiler_params=pltpu.CompilerParams(use_tc_tiling_on_sc=True, needs_layout_passes=False),
    )
    def gather_kernel(tab_hbm, idx_hbm, out_hbm, idx_v, rows_v, obuf, gsem, ssem):
        wid = lax.axis_index("s") * NC + lax.axis_index("c")
        n0 = pl.multiple_of(wid * _NB, _NB)
        pltpu.sync_copy(idx_hbm.at[:, pl.ds(n0, _NB)], idx_v)

        def fire_gather(r, b):
            pltpu.async_copy(
                tab_hbm.at[idx_v.at[r]], rows_v.at[b], gsem.at[b]
            )

        def wait_gather(b):
            pltpu.make_async_copy(
                tab_hbm.at[pl.ds(0, _NB)], rows_v.at[b], gsem.at[b]
            ).wait()

        def fire_store(r, b):
            pltpu.async_copy(
                obuf.at[b], out_hbm.at[r].at[:, pl.ds(n0, _NB)], ssem.at[b]
            )

        def wait_store(b):
            pltpu.make_async_copy(
                obuf.at[b], out_hbm.at[0].at[:, pl.ds(0, _NB)], ssem.at[b]
            ).wait()

        def flush(r, b):
            _transpose_16(rows_v.at[b], obuf.at[b], _NB, scatter=False)
            fire_store(r, b)

        def step(k, carry):
            fire_gather(2 * k, 0)

            @pl.when(k > 0)
            def _():
                wait_gather(1)

            @pl.when(k > 1)
            def _():
                wait_store(1)

            @pl.when(k > 0)
            def _():
                flush(2 * k - 1, 1)

            fire_gather(2 * k + 1, 1)
            wait_gather(0)

            @pl.when(k > 0)
            def _():
                wait_store(0)

            flush(2 * k, 0)
            return carry

        lax.fori_loop(0, S // _NBUF, step, 0)
        wait_gather(1)
        wait_store(1)
        flush(S - 1, 1)
        wait_store(0)
        wait_store(1)

    return gather_kernel


@functools.partial(jax.jit, static_argnums=(2, 3, 4))
def _embed(table, idx, V, N, S):
    table_pad = _make_format_table(V)(table.T)
    out_t = _make_gather(N, S)(table_pad, idx.T)
    return out_t.transpose(2, 0, 1)


def kernel(pre_embedding, preembed_mask, embed_table):
    N, S = pre_embedding.shape
    V = embed_table.shape[0]
    out = _embed(embed_table, pre_embedding, V, N, S)
    return out, preembed_mask


# speedup vs baseline: 3.2493x; 3.1654x over previous
import functools

import jax
import jax.numpy as jnp
from jax import lax
from jax.experimental import pallas as pl
from jax.experimental.pallas import tpu as pltpu
from jax.experimental.pallas import tpu_sc as plsc

EMBED_DIM = 64
PAD_DIM = 128
NC = 2
NS = 16
NW = NC * NS
LANES = 16

_CW = 256
_NB = 128
_NBUF = 2


def _transpose_16(fm_ref, rm_ref, n, to_rm):
    steps = n // LANES

    def body(kk, carry):
        i = lax.iota(jnp.int32, LANES)
        nn = kk * LANES + i
        for d0 in range(0, EMBED_DIM, LANES):
            loaded = []
            for d in range(LANES):
                c = d0 + ((d + i) & (LANES - 1))
                if to_rm:
                    loaded.append((c, plsc.load_gather(fm_ref, [c, nn])))
                else:
                    loaded.append((c, plsc.load_gather(rm_ref, [nn, c])))
            for c, v in loaded:
                if to_rm:
                    plsc.store_scatter(rm_ref, [nn, c], v)
                else:
                    plsc.store_scatter(fm_ref, [c, nn], v)
        return carry

    lax.fori_loop(0, steps, body, 0)


def _make_format_table(V):
    n_full = V // _CW
    tail = V - n_full * _CW
    n_iters = (n_full + NW - 1) // NW

    mesh = plsc.VectorSubcoreMesh(core_axis_name="c", subcore_axis_name="s")

    @functools.partial(
        pl.kernel,
        mesh=mesh,
        out_type=jax.ShapeDtypeStruct((V, PAD_DIM), jnp.float32),
        scratch_types=[
            pltpu.VMEM((EMBED_DIM, _CW), jnp.float32),
            pltpu.VMEM((_CW, PAD_DIM), jnp.float32),
            pltpu.VMEM((EMBED_DIM, tail), jnp.float32),
        ],
        compiler_params=pltpu.CompilerParams(use_tc_tiling_on_sc=True, needs_layout_passes=False),
    )
    def tpose_kernel(tt_hbm, out_hbm, tsrc, tdst, ttail):
        wid = lax.axis_index("s") * NC + lax.axis_index("c")

        def chunk(k, carry):
            c = k * NW + wid

            @pl.when(c < n_full)
            def _():
                c0 = pl.multiple_of(c * _CW, _CW)
                pltpu.sync_copy(tt_hbm.at[:, pl.ds(c0, _CW)], tsrc)
                _transpose_16(tsrc, tdst, _CW, to_rm=True)
                pltpu.sync_copy(tdst, out_hbm.at[pl.ds(c0, _CW)])
            return carry

        lax.fori_loop(0, n_iters, chunk, 0)

        @pl.when(wid == 0)
        def _():
            c0 = n_full * _CW
            pltpu.sync_copy(tt_hbm.at[:, pl.ds(c0, tail)], ttail)
            _transpose_16(ttail, tdst, tail, to_rm=True)
            pltpu.sync_copy(
                tdst.at[pl.ds(0, tail)], out_hbm.at[pl.ds(c0, tail)]
            )

    return tpose_kernel


def _make_gather(N, S):
    mesh = plsc.VectorSubcoreMesh(core_axis_name="c", subcore_axis_name="s")

    @functools.partial(
        pl.kernel,
        mesh=mesh,
        out_type=jax.ShapeDtypeStruct((S, EMBED_DIM, N), jnp.float32),
        scratch_types=[
            pltpu.VMEM((S, _NB), jnp.int32),
            pltpu.VMEM((_NBUF, _NB, PAD_DIM), jnp.float32),
            pltpu.VMEM((_NBUF, EMBED_DIM, _NB), jnp.float32),
            pltpu.SemaphoreType.DMA((_NBUF,)),
            pltpu.SemaphoreType.DMA((_NBUF,)),
        ],
        compiler_params=pltpu.CompilerParams(use_tc_tiling_on_sc=True, needs_layout_passes=False),
    )
    def gather_kernel(tab_hbm, idx_hbm, out_hbm, idx_v, rows_v, obuf, gsem, ssem):
        wid = lax.axis_index("s") * NC + lax.axis_index("c")
        n0 = pl.multiple_of(wid * _NB, _NB)
        pltpu.sync_copy(idx_hbm.at[:, pl.ds(n0, _NB)], idx_v)

        def fire_gather(r, b):
            pltpu.async_copy(
                tab_hbm.at[idx_v.at[r]], rows_v.at[b], gsem.at[b]
            )

        def wait_gather(b):
            pltpu.make_async_copy(
                tab_hbm.at[pl.ds(0, _NB)], rows_v.at[b], gsem.at[b]
            ).wait()

        def fire_store(r, b):
            pltpu.async_copy(
                obuf.at[b], out_hbm.at[r].at[:, pl.ds(n0, _NB)], ssem.at[b]
            )

        def wait_store(b):
            pltpu.make_async_copy(
                obuf.at[b], out_hbm.at[0].at[:, pl.ds(0, _NB)], ssem.at[b]
            ).wait()

        def flush(r, b):
            _transpose_16(obuf.at[b], rows_v.at[b], _NB, to_rm=False)
            fire_store(r, b)

        def step(k, carry):
            fire_gather(2 * k, 0)

            @pl.when(k > 0)
            def _():
                wait_gather(1)

            @pl.when(k > 1)
            def _():
                wait_store(1)

            @pl.when(k > 0)
            def _():
                flush(2 * k - 1, 1)

            fire_gather(2 * k + 1, 1)
            wait_gather(0)

            @pl.when(k > 0)
            def _():
                wait_store(0)

            flush(2 * k, 0)
            return carry

        lax.fori_loop(0, S // _NBUF, step, 0)
        wait_gather(1)
        wait_store(1)
        flush(S - 1, 1)
        wait_store(0)
        wait_store(1)

    return gather_kernel


@functools.partial(jax.jit, static_argnums=(2, 3, 4))
def _embed(table, idx, V, N, S):
    table_pad = _make_format_table(V)(table.T)
    out_t = _make_gather(N, S)(table_pad, idx.T)
    return out_t.transpose(2, 0, 1)


def kernel(pre_embedding, preembed_mask, embed_table):
    N, S = pre_embedding.shape
    V = embed_table.shape[0]
    out = _embed(embed_table, pre_embedding, V, N, S)
    return out, preembed_mask


# speedup vs baseline: 4.5961x; 1.4145x over previous
import functools

import jax
import jax.numpy as jnp
from jax import lax
from jax.experimental import pallas as pl
from jax.experimental.pallas import tpu as pltpu
from jax.experimental.pallas import tpu_sc as plsc

EMBED_DIM = 64
PAD_DIM = 128
NC = 2
NS = 16
NW = NC * NS
LANES = 16

_CW = 256
_NB = 128
_NBUF = 2


def _transpose_16(fm_ref, rm_ref, n, to_rm):
    steps = n // LANES

    def body(kk, carry):
        i = lax.iota(jnp.int32, LANES)
        nn = kk * LANES + i
        for d0 in range(0, EMBED_DIM, LANES):
            loaded = []
            for d in range(LANES):
                c = d0 + ((d + i) & (LANES - 1))
                if to_rm:
                    loaded.append((c, plsc.load_gather(fm_ref, [c, nn])))
                else:
                    loaded.append((c, plsc.load_gather(rm_ref, [nn, c])))
            for c, v in loaded:
                if to_rm:
                    plsc.store_scatter(rm_ref, [nn, c], v)
                else:
                    plsc.store_scatter(fm_ref, [c, nn], v)
        return carry

    lax.fori_loop(0, steps, body, 0)


def _make_format_table(V):
    n_full = V // _CW
    tail = V - n_full * _CW
    n_iters = (n_full + NW - 1) // NW

    mesh = plsc.VectorSubcoreMesh(core_axis_name="c", subcore_axis_name="s")

    @functools.partial(
        pl.kernel,
        mesh=mesh,
        out_type=jax.ShapeDtypeStruct((V, PAD_DIM), jnp.float32),
        scratch_types=[
            pltpu.VMEM((2, EMBED_DIM, _CW), jnp.float32),
            pltpu.VMEM((2, _CW, PAD_DIM), jnp.float32),
            pltpu.VMEM((EMBED_DIM, tail), jnp.float32),
            pltpu.SemaphoreType.DMA((2,)),
            pltpu.SemaphoreType.DMA((2,)),
        ],
        compiler_params=pltpu.CompilerParams(use_tc_tiling_on_sc=True, needs_layout_passes=False),
    )
    def tpose_kernel(tt_hbm, out_hbm, tsrc, tdst, ttail, rsem, wsem):
        wid = lax.axis_index("s") * NC + lax.axis_index("c")

        def fire_read(k, b):
            c = k * NW + wid

            @pl.when(c < n_full)
            def _():
                c0 = pl.multiple_of(c * _CW, _CW)
                pltpu.async_copy(
                    tt_hbm.at[:, pl.ds(c0, _CW)], tsrc.at[b], rsem.at[b]
                )

        def wait_read(b):
            pltpu.make_async_copy(
                tt_hbm.at[:, pl.ds(0, _CW)], tsrc.at[b], rsem.at[b]
            ).wait()

        def fire_write(k, b):
            c = k * NW + wid
            c0 = pl.multiple_of(c * _CW, _CW)
            pltpu.async_copy(tdst.at[b], out_hbm.at[pl.ds(c0, _CW)], wsem.at[b])

        def wait_write(b):
            pltpu.make_async_copy(
                tdst.at[b], out_hbm.at[pl.ds(0, _CW)], wsem.at[b]
            ).wait()

        fire_read(0, 0)

        def step(k2, carry):
            for b in range(2):
                k = k2 * 2 + b
                c = k * NW + wid

                @pl.when((k > 1) & ((k - 2) * NW + wid < n_full))
                def _():
                    wait_write(b)

                @pl.when(c < n_full)
                def _():
                    wait_read(b)
                    fire_read(k + 1, 1 - b)
                    _transpose_16(tsrc.at[b], tdst.at[b], _CW, to_rm=True)
                    fire_write(k, b)
            return carry

        n_pairs = (n_iters + 1) // 2
        lax.fori_loop(0, n_pairs, step, 0)
        for j in (2 * n_pairs - 2, 2 * n_pairs - 1):
            cj = j * NW + wid

            @pl.when(cj < n_full)
            def _():
                wait_write(j % 2)

        @pl.when(wid == 0)
        def _():
            c0 = n_full * _CW
            pltpu.sync_copy(tt_hbm.at[:, pl.ds(c0, tail)], ttail)
            _transpose_16(ttail, tdst.at[0], tail, to_rm=True)
            pltpu.sync_copy(
                tdst.at[0].at[pl.ds(0, tail)], out_hbm.at[pl.ds(c0, tail)]
            )

    return tpose_kernel


def _make_gather(N, S):
    mesh = plsc.VectorSubcoreMesh(core_axis_name="c", subcore_axis_name="s")

    @functools.partial(
        pl.kernel,
        mesh=mesh,
        out_type=jax.ShapeDtypeStruct((S, EMBED_DIM, N), jnp.float32),
        scratch_types=[
            pltpu.VMEM((S, _NB), jnp.int32),
            pltpu.VMEM((_NBUF, _NB, PAD_DIM), jnp.float32),
            pltpu.VMEM((_NBUF, EMBED_DIM, _NB), jnp.float32),
            pltpu.SemaphoreType.DMA((_NBUF,)),
            pltpu.SemaphoreType.DMA((_NBUF,)),
        ],
        compiler_params=pltpu.CompilerParams(use_tc_tiling_on_sc=True, needs_layout_passes=False),
    )
    def gather_kernel(tab_hbm, idx_hbm, out_hbm, idx_v, rows_v, obuf, gsem, ssem):
        wid = lax.axis_index("s") * NC + lax.axis_index("c")
        n0 = pl.multiple_of(wid * _NB, _NB)
        pltpu.sync_copy(idx_hbm.at[:, pl.ds(n0, _NB)], idx_v)

        def fire_gather(r, b):
            pltpu.async_copy(
                tab_hbm.at[idx_v.at[r]], rows_v.at[b], gsem.at[b]
            )

        def wait_gather(b):
            pltpu.make_async_copy(
                tab_hbm.at[pl.ds(0, _NB)], rows_v.at[b], gsem.at[b]
            ).wait()

        def fire_store(r, b):
            pltpu.async_copy(
                obuf.at[b], out_hbm.at[r].at[:, pl.ds(n0, _NB)], ssem.at[b]
            )

        def wait_store(b):
            pltpu.make_async_copy(
                obuf.at[b], out_hbm.at[0].at[:, pl.ds(0, _NB)], ssem.at[b]
            ).wait()

        def flush(r, b):
            _transpose_16(obuf.at[b], rows_v.at[b], _NB, to_rm=False)
            fire_store(r, b)

        def step(k, carry):
            fire_gather(2 * k, 0)

            @pl.when(k > 0)
            def _():
                wait_gather(1)

            @pl.when(k > 1)
            def _():
                wait_store(1)

            @pl.when(k > 0)
            def _():
                flush(2 * k - 1, 1)

            fire_gather(2 * k + 1, 1)
            wait_gather(0)

            @pl.when(k > 0)
            def _():
                wait_store(0)

            flush(2 * k, 0)
            return carry

        lax.fori_loop(0, S // _NBUF, step, 0)
        wait_gather(1)
        wait_store(1)
        flush(S - 1, 1)
        wait_store(0)
        wait_store(1)

    return gather_kernel


@functools.partial(jax.jit, static_argnums=(2, 3, 4))
def _embed(table, idx, V, N, S):
    table_pad = _make_format_table(V)(table.T)
    out_t = _make_gather(N, S)(table_pad, idx.T)
    return out_t.transpose(2, 0, 1)


def kernel(pre_embedding, preembed_mask, embed_table):
    N, S = pre_embedding.shape
    V = embed_table.shape[0]
    out = _embed(embed_table, pre_embedding, V, N, S)
    return out, preembed_mask
